# baseline (speedup 1.0000x reference)
import functools

import jax
import jax.numpy as jnp
from jax import lax
from jax.experimental import pallas as pl
from jax.experimental.pallas import tpu as pltpu
from jax.experimental.pallas import tpu_sc as plsc

VOCAB = 1000000
EMB_DIM = 64
NUM_CLASS = 7
BATCH = 16384

WPR = EMB_DIM // 2
NC = 2
NS = 16
L = 16
NW = NC * NS
BPW = BATCH // NW
IDX_CHUNK = 128
N_IDX_CHUNKS = BPW // IDX_CHUNK
GPC = 4
N_CCHUNK = BPW // (GPC * L)


def _sc_call(node, table_p, w_splat, b_splat):
    mesh = plsc.VectorSubcoreMesh(core_axis_name="c", subcore_axis_name="s")

    @functools.partial(
        pl.kernel,
        mesh=mesh,
        compiler_params=pltpu.CompilerParams(
            needs_layout_passes=False, use_tc_tiling_on_sc=False
        ),
        out_type=jax.ShapeDtypeStruct((BATCH, NUM_CLASS), jnp.float32),
        scratch_types=[
            pltpu.VMEM((N_IDX_CHUNKS, IDX_CHUNK), jnp.int32),
            pltpu.VMEM((BPW, WPR), jnp.int32),
            pltpu.VMEM((NUM_CLASS * EMB_DIM, L), jnp.float32),
            pltpu.VMEM((8, L), jnp.float32),
            pltpu.VMEM((BPW, NUM_CLASS), jnp.float32),
            pltpu.SemaphoreType.DMA,
        ],
    )
    def k(node_h, table_h, w_h, b_h, out_h, idx_v, rows_v, w_v, b_v, out_v, sem):
        wid = lax.axis_index("s") * NC + lax.axis_index("c")
        base = wid * BPW

        for j in range(N_IDX_CHUNKS):
            pltpu.sync_copy(
                node_h.at[pl.ds(base + j * IDX_CHUNK, IDX_CHUNK)], idx_v.at[j]
            )
        pltpu.sync_copy(w_h, w_v)
        pltpu.sync_copy(b_h, b_v)

        copies = [
            pltpu.async_copy(
                table_h.at[idx_v.at[j]],
                rows_v.at[pl.ds(j * IDX_CHUNK, IDX_CHUNK)],
                sem,
            )
            for j in range(N_IDX_CHUNKS)
        ]
        for c in copies:
            c.wait()

        iota = lax.iota(jnp.int32, L)
        himask = jnp.full((L,), -65536, jnp.int32)

        def chunk_body(ch, carry):
            nbase = ch * GPC * L
            row_idx = [
                jnp.full((L,), nbase + q * L, jnp.int32) + iota for q in range(GPC)
            ]

            def d_body(d2, accs):
                col = jnp.full((L,), d2, jnp.int32)
                ws = [plsc.load_gather(rows_v, [row_idx[q], col]) for q in range(GPC)]
                los = [plsc.bitcast(w << 16, jnp.float32) for w in ws]
                his = [plsc.bitcast(w & himask, jnp.float32) for w in ws]
                out = []
                for c in range(NUM_CLASS):
                    wlo = w_v[c * EMB_DIM + 2 * d2]
                    whi = w_v[c * EMB_DIM + 2 * d2 + 1]
                    for q in range(GPC):
                        out.append(accs[c * GPC + q] + los[q] * wlo + his[q] * whi)
                return tuple(out)

            init = tuple(b_v[c] for c in range(NUM_CLASS) for _ in range(GPC))
            accs = lax.fori_loop(0, WPR, d_body, init)

            for c in range(NUM_CLASS):
                ccol = jnp.full((L,), c, jnp.int32)
                for q in range(GPC):
                    plsc.store_scatter(out_v, [row_idx[q], ccol], accs[c * GPC + q])
            return carry

        lax.fori_loop(0, N_CCHUNK, chunk_body, 0)

        pltpu.sync_copy(out_v, out_h.at[pl.ds(base, BPW)])

    return k(node, table_p, w_splat, b_splat)


def kernel(node, emb_table, fc_w, fc_b):
    tbl_p = jax.lax.bitcast_convert_type(
        emb_table.astype(jnp.bfloat16).reshape(VOCAB, WPR, 2), jnp.int32
    )
    w_splat = jnp.broadcast_to(
        fc_w.reshape(NUM_CLASS * EMB_DIM, 1), (NUM_CLASS * EMB_DIM, L)
    )
    b_pad = jnp.concatenate([fc_b, jnp.zeros((1,), jnp.float32)])
    b_splat = jnp.broadcast_to(b_pad.reshape(8, 1), (8, L))
    return _sc_call(node, tbl_p, w_splat, b_splat)


# speedup vs baseline: 2.6150x; 2.6150x over previous
import functools

import jax
import jax.numpy as jnp
from jax import lax
from jax.experimental import pallas as pl
from jax.experimental.pallas import tpu as pltpu
from jax.experimental.pallas import tpu_sc as plsc

VOCAB = 1000000
EMB_DIM = 64
NUM_CLASS = 7
BATCH = 16384

NC = 2
NS = 16
L = 16
NW = NC * NS
BPW = BATCH // NW
IDX_CHUNK = 128
N_IDX_CHUNKS = BPW // IDX_CHUNK
GPC = 4
N_CCHUNK = BPW // (GPC * L)
WROW = 2 * EMB_DIM


def _sc_call(node_pair, node_off, table2, w_flat, b_flat):
    mesh = plsc.VectorSubcoreMesh(core_axis_name="c", subcore_axis_name="s")

    @functools.partial(
        pl.kernel,
        mesh=mesh,
        compiler_params=pltpu.CompilerParams(
            needs_layout_passes=False, use_tc_tiling_on_sc=True
        ),
        out_type=jax.ShapeDtypeStruct((BATCH * NUM_CLASS,), jnp.float32),
        scratch_types=[
            pltpu.VMEM((BPW,), jnp.int32),
            pltpu.VMEM((BPW,), jnp.int32),
            pltpu.VMEM((BPW, WROW), jnp.float32),
            pltpu.VMEM((NUM_CLASS * EMB_DIM * L,), jnp.float32),
            pltpu.VMEM((8 * L,), jnp.float32),
            pltpu.VMEM((BPW * NUM_CLASS,), jnp.float32),
            pltpu.SemaphoreType.DMA,
        ],
    )
    def k(pair_h, off_h, table_h, w_h, b_h, out_h, pair_v, off_v, rows_v, w_v,
          b_v, out_v, sem):
        wid = lax.axis_index("s") * NC + lax.axis_index("c")
        base = wid * BPW

        pltpu.sync_copy(pair_h.at[pl.ds(base, BPW)], pair_v)
        pltpu.sync_copy(off_h.at[pl.ds(base, BPW)], off_v)
        pltpu.sync_copy(w_h, w_v)
        pltpu.sync_copy(b_h, b_v)

        copies = [
            pltpu.async_copy(
                table_h.at[pair_v.at[pl.ds(j * IDX_CHUNK, IDX_CHUNK)]],
                rows_v.at[pl.ds(j * IDX_CHUNK, IDX_CHUNK), :],
                sem,
            )
            for j in range(N_IDX_CHUNKS)
        ]
        for c in copies:
            c.wait()

        iota = lax.iota(jnp.int32, L)

        def chunk_body(ch, carry):
            nbase = ch * GPC * L
            row_idx = [
                jnp.full((L,), nbase + q * L, jnp.int32) + iota for q in range(GPC)
            ]
            offs = [
                plsc.load_gather(off_v, [row_idx[q]]) for q in range(GPC)
            ]

            def d_body(d, accs):
                cols = [offs[q] + d for q in range(GPC)]
                es = [
                    plsc.load_gather(rows_v, [row_idx[q], cols[q]])
                    for q in range(GPC)
                ]
                out = []
                for c in range(NUM_CLASS):
                    widx = jnp.full((L,), (c * EMB_DIM) * L, jnp.int32) + d * L + iota
                    wv = plsc.load_gather(w_v, [widx])
                    for q in range(GPC):
                        out.append(accs[c * GPC + q] + es[q] * wv)
                return tuple(out)

            init = tuple(
                plsc.load_gather(b_v, [jnp.full((L,), c * L, jnp.int32) + iota])
                for c in range(NUM_CLASS)
                for _ in range(GPC)
            )
            accs = lax.fori_loop(0, EMB_DIM, d_body, init)

            for c in range(NUM_CLASS):
                for q in range(GPC):
                    oidx = row_idx[q] * NUM_CLASS + c
                    plsc.store_scatter(out_v, [oidx], accs[c * GPC + q])
            return carry

        lax.fori_loop(0, N_CCHUNK, chunk_body, 0)

        pltpu.sync_copy(out_v, out_h.at[pl.ds(base * NUM_CLASS, BPW * NUM_CLASS)])

    return k(node_pair, node_off, table2, w_flat, b_flat)


def kernel(node, emb_table, fc_w, fc_b):
    table2 = emb_table.reshape(VOCAB // 2, WROW)
    node_pair = node >> 1
    node_off = (node & 1) * EMB_DIM
    w_flat = jnp.broadcast_to(
        fc_w.reshape(NUM_CLASS * EMB_DIM, 1), (NUM_CLASS * EMB_DIM, L)
    ).reshape(-1)
    b_pad = jnp.concatenate([fc_b, jnp.zeros((1,), jnp.float32)])
    b_flat = jnp.broadcast_to(b_pad.reshape(8, 1), (8, L)).reshape(-1)
    out = _sc_call(node_pair, node_off, table2, w_flat, b_flat)
    return out.reshape(BATCH, NUM_CLASS)
